# baseline (speedup 1.0000x reference)
import jax
import jax.numpy as jnp
from jax import lax
from jax.experimental import pallas as pl
from jax.experimental.pallas import tpu as pltpu
from jax.experimental.pallas import tpu_sc as plsc

_WINDOW = 512
_N_FRAMES = 18750
_BLK = 592
_N_TILES = 32
_N_PAD = _BLK * _N_TILES
_GPT = _BLK // 16
_DUMP = _N_PAD
_TAILDUP = _N_PAD + 16
_SRC_SZ = _N_PAD + 32
_LAST_ROWS = _N_FRAMES - (_N_TILES - 1) * _BLK
_TAIL0 = _N_FRAMES - 16
_LTAIL0 = _TAIL0 - (_N_TILES - 1) * _BLK


def _score_body(x_ref, w_ref, b_ref, pdest_ref, nsp_ref, carry_ref):
    k = pl.program_id(0)
    frames = x_ref[...]
    w = w_ref[...]
    logits = jnp.dot(frames, w, preferred_element_type=jnp.float32)
    logits = logits + b_ref[0]
    row = lax.broadcasted_iota(jnp.int32, (_BLK, 1), 0) + k * _BLK
    flag = jnp.where((logits > 0.0) & (row < _N_FRAMES), 1.0, 0.0)
    i = lax.broadcasted_iota(jnp.int32, (_BLK, _BLK), 0)
    j = lax.broadcasted_iota(jnp.int32, (_BLK, _BLK), 1)
    tri = jnp.where(i >= j, 1.0, 0.0)
    csum = jnp.dot(tri, flag, preferred_element_type=jnp.float32)

    @pl.when(k == 0)
    def _():
        carry_ref[0, 0] = 0.0

    carry = carry_ref[0, 0]
    dest = jnp.where(flag > 0.0, carry + csum, 0.0)
    pdest_ref[...] = dest.astype(jnp.int32).reshape(1, 1, _BLK)
    total = carry + jnp.max(csum)
    nsp_ref[...] = jnp.full((1, 1, 128), total, jnp.float32).astype(jnp.int32)
    carry_ref[0, 0] = total


def _score(x2d, w, b):
    pdest, nsp = pl.pallas_call(
        _score_body,
        grid=(_N_TILES,),
        in_specs=[
            pl.BlockSpec((_BLK, _WINDOW), lambda k: (k, 0)),
            pl.BlockSpec((_WINDOW, 1), lambda k: (0, 0)),
            pl.BlockSpec(memory_space=pltpu.SMEM),
        ],
        out_specs=[
            pl.BlockSpec((1, 1, _BLK), lambda k: (k, 0, 0)),
            pl.BlockSpec((1, 1, 128), lambda k: (0, 0, 0)),
        ],
        out_shape=[
            jax.ShapeDtypeStruct((_N_TILES, 1, _BLK), jnp.int32),
            jax.ShapeDtypeStruct((1, 1, 128), jnp.int32),
        ],
        scratch_shapes=[pltpu.SMEM((1, 1), jnp.float32)],
    )(x2d, w.reshape(_WINDOW, 1), b.reshape(1))
    return pdest.reshape(_N_PAD), nsp.reshape(128)


def _scatter_body(pdest_hbm, src_hbm, pd_v, pos_v, pos2_v, val_v, sem):
    cid = lax.axis_index("c")
    sid = lax.axis_index("s")
    wid = sid * 2 + cid
    base = wid * _BLK

    pltpu.sync_copy(pdest_hbm.at[pl.ds(base, _BLK)], pd_v)
    lanes = lax.iota(jnp.int32, 16)

    def _g(g, c):
        p = pd_v[pl.ds(g * 16, 16)]
        m = p > 0
        d = p - 1
        pos_v[pl.ds(g * 16, 16)] = jnp.where(m, d, _DUMP + lanes)
        m2 = jnp.logical_and(m, d >= _TAIL0)
        pos2_v[pl.ds(g * 16, 16)] = jnp.where(m2, _TAILDUP + (d - _TAIL0),
                                              _DUMP + lanes)
        val_v[pl.ds(g * 16, 16)] = base + g * 16 + lanes
        return c

    lax.fori_loop(0, _GPT, _g, 0)
    pltpu.async_copy(val_v, src_hbm.at[pos_v], sem).wait()
    pltpu.async_copy(val_v, src_hbm.at[pos2_v], sem).wait()


def _scatter(pdest):
    mesh = plsc.VectorSubcoreMesh(core_axis_name="c", subcore_axis_name="s",
                                  num_cores=2, num_subcores=16)
    f = pl.kernel(
        _scatter_body,
        out_type=jax.ShapeDtypeStruct((_SRC_SZ,), jnp.int32),
        mesh=mesh,
        scratch_types=[
            pltpu.VMEM((_BLK,), jnp.int32),
            pltpu.VMEM((_BLK,), jnp.int32),
            pltpu.VMEM((_BLK,), jnp.int32),
            pltpu.VMEM((_BLK,), jnp.int32),
            pltpu.SemaphoreType.DMA,
        ],
    )
    return f(pdest)


def _gather_body(x_hbm, src_hbm, nsp_hbm, out_hbm,
                 src_v, nsp_v, tidx_v, ridx_v, data_v, zero_v, sem):
    cid = lax.axis_index("c")
    sid = lax.axis_index("s")
    wid = sid * 2 + cid
    a = wid * _BLK
    lanes = lax.iota(jnp.int32, 16)
    zf = jnp.zeros((16,), jnp.float32)

    pltpu.sync_copy(nsp_hbm.at[pl.ds(0, 16)], nsp_v)
    pltpu.sync_copy(src_hbm.at[pl.ds(a, _BLK)], src_v)
    nsp = nsp_v[pl.ds(0, 16)][0]
    count = jnp.clip(nsp - a, 0, _BLK)

    def _zb(t, c):
        zero_v[t // 32, pl.ds((t % 32) * 16, 16)] = zf
        return c

    lax.fori_loop(0, 512, _zb, 0)

    def _chunk(c, carry):
        v = count - c * 16

        @pl.when(v > 0)
        def _():
            raw = src_v[pl.ds(c * 16, 16)]
            safe = jnp.where(lanes < v, jnp.clip(raw, 0, _N_FRAMES - 1), 0)
            pltpu.async_copy(x_hbm.at[safe], data_v, sem).wait()

            @pl.when(v < 16)
            def _():
                def _zt(t, cc):
                    data_v[t // 32, pl.ds((t % 32) * 16, 16)] = zf
                    return cc
                lax.fori_loop(v * 32, 512, _zt, 0)

            pltpu.sync_copy(data_v, out_hbm.at[pl.ds(a + c * 16, 16)])

        @pl.when(v <= 0)
        def _():
            pltpu.sync_copy(zero_v, out_hbm.at[pl.ds(a + c * 16, 16)])

        return carry

    n_full = jnp.where(wid == _N_TILES - 1, _LAST_ROWS // 16, _GPT)
    lax.fori_loop(0, n_full, _chunk, 0)

    @pl.when(wid == _N_TILES - 1)
    def _():
        pltpu.sync_copy(src_hbm.at[pl.ds(_TAILDUP, 16)], tidx_v)
        ridx_v[...] = _TAIL0 + lanes
        v2 = count - _LTAIL0

        @pl.when(v2 > 0)
        def _():
            raw = tidx_v[pl.ds(0, 16)]
            safe = jnp.where(lanes < v2, jnp.clip(raw, 0, _N_FRAMES - 1), 0)
            pltpu.async_copy(x_hbm.at[safe], data_v, sem).wait()

        def _zt(t, cc):
            data_v[t // 32, pl.ds((t % 32) * 16, 16)] = zf
            return cc

        lax.fori_loop(jnp.clip(v2, 0, 16) * 32, 512, _zt, 0)
        pltpu.async_copy(data_v, out_hbm.at[ridx_v], sem).wait()


def _gather(x2d, src, nsp):
    mesh = plsc.VectorSubcoreMesh(core_axis_name="c", subcore_axis_name="s",
                                  num_cores=2, num_subcores=16)
    f = pl.kernel(
        _gather_body,
        out_type=jax.ShapeDtypeStruct((_N_FRAMES, _WINDOW), jnp.float32),
        mesh=mesh,
        scratch_types=[
            pltpu.VMEM((_BLK,), jnp.int32),
            pltpu.VMEM((16,), jnp.int32),
            pltpu.VMEM((16,), jnp.int32),
            pltpu.VMEM((16,), jnp.int32),
            pltpu.VMEM((16, _WINDOW), jnp.float32),
            pltpu.VMEM((16, _WINDOW), jnp.float32),
            pltpu.SemaphoreType.DMA,
        ],
    )
    return f(x2d, src, nsp)


def kernel(x, W, b):
    x2d = x[: _N_FRAMES * _WINDOW].reshape(_N_FRAMES, _WINDOW)
    pdest, nsp = _score(x2d, W, b)
    src = _scatter(pdest)
    out = jnp.zeros(_N_FRAMES * _WINDOW, jnp.float32)
    out = out.at[:_SRC_SZ].set(src.astype(jnp.float32))
    return out


# speedup vs baseline: 14.1318x; 14.1318x over previous
import jax
import jax.numpy as jnp
from jax import lax
from jax.experimental import pallas as pl
from jax.experimental.pallas import tpu as pltpu
from jax.experimental.pallas import tpu_sc as plsc

_WINDOW = 512
_N_FRAMES = 18750
_BLK = 592
_N_TILES = 32
_N_PAD = _BLK * _N_TILES
_GPT = _BLK // 16
_TAILDUP = _N_PAD
_DUMP0 = _N_PAD + 16
_SRC_SZ = _DUMP0 + _N_PAD
_LAST_ROWS = _N_FRAMES - (_N_TILES - 1) * _BLK
_TAIL0 = _N_FRAMES - 16
_LTAIL0 = _TAIL0 - (_N_TILES - 1) * _BLK


def _score_body(x_ref, w_ref, b_ref, pdest_ref, nsp_ref, carry_ref):
    k = pl.program_id(0)
    frames = x_ref[...]
    w = w_ref[...]
    logits = jnp.dot(frames, w, preferred_element_type=jnp.float32)
    logits = logits + b_ref[0]
    row = lax.broadcasted_iota(jnp.int32, (_BLK, 1), 0) + k * _BLK
    flag = jnp.where((logits > 0.0) & (row < _N_FRAMES), 1.0, 0.0)
    i = lax.broadcasted_iota(jnp.int32, (_BLK, _BLK), 0)
    j = lax.broadcasted_iota(jnp.int32, (_BLK, _BLK), 1)
    tri = jnp.where(i >= j, 1.0, 0.0)
    csum = jnp.dot(tri, flag, preferred_element_type=jnp.float32)

    @pl.when(k == 0)
    def _():
        carry_ref[0, 0] = 0.0

    carry = carry_ref[0, 0]
    dest = jnp.where(flag > 0.0, carry + csum, 0.0)
    pdest_ref[...] = dest.astype(jnp.int32).reshape(1, 1, _BLK)
    total = carry + jnp.max(csum)
    nsp_ref[...] = jnp.full((1, 1, 128), total, jnp.float32).astype(jnp.int32)
    carry_ref[0, 0] = total


def _score(x2d, w, b):
    pdest, nsp = pl.pallas_call(
        _score_body,
        grid=(_N_TILES,),
        in_specs=[
            pl.BlockSpec((_BLK, _WINDOW), lambda k: (k, 0)),
            pl.BlockSpec((_WINDOW, 1), lambda k: (0, 0)),
            pl.BlockSpec(memory_space=pltpu.SMEM),
        ],
        out_specs=[
            pl.BlockSpec((1, 1, _BLK), lambda k: (k, 0, 0)),
            pl.BlockSpec((1, 1, 128), lambda k: (0, 0, 0)),
        ],
        out_shape=[
            jax.ShapeDtypeStruct((_N_TILES, 1, _BLK), jnp.int32),
            jax.ShapeDtypeStruct((1, 1, 128), jnp.int32),
        ],
        scratch_shapes=[pltpu.SMEM((1, 1), jnp.float32)],
    )(x2d, w.reshape(_WINDOW, 1), b.reshape(1))
    return pdest.reshape(_N_PAD), nsp.reshape(128)


def _scatter_body(pdest_hbm, src_hbm, pd_v, pos_v, pos2_v, val_v, sem):
    cid = lax.axis_index("c")
    sid = lax.axis_index("s")
    wid = sid * 2 + cid
    base = wid * _BLK

    pltpu.sync_copy(pdest_hbm.at[pl.ds(base, _BLK)], pd_v)
    lanes = lax.iota(jnp.int32, 16)

    def _g(g, m2v):
        p = pd_v[pl.ds(g * 16, 16)]
        m = p > 0
        d = p - 1
        i = base + g * 16 + lanes
        pos_v[pl.ds(g * 16, 16)] = jnp.where(m, d, _DUMP0 + i)
        m2 = jnp.logical_and(m, d >= _TAIL0)
        pos2_v[pl.ds(g * 16, 16)] = jnp.where(m2, _TAILDUP + (d - _TAIL0),
                                              _DUMP0 + i)
        val_v[pl.ds(g * 16, 16)] = i
        return m2v

    lax.fori_loop(0, _GPT, _g, 0)
    pltpu.async_copy(val_v, src_hbm.at[pos_v], sem).wait()
    pltpu.async_copy(val_v, src_hbm.at[pos2_v], sem).wait()


def _scatter(pdest):
    mesh = plsc.VectorSubcoreMesh(core_axis_name="c", subcore_axis_name="s",
                                  num_cores=2, num_subcores=16)
    f = pl.kernel(
        _scatter_body,
        out_type=jax.ShapeDtypeStruct((_SRC_SZ,), jnp.int32),
        mesh=mesh,
        scratch_types=[
            pltpu.VMEM((_BLK,), jnp.int32),
            pltpu.VMEM((_BLK,), jnp.int32),
            pltpu.VMEM((_BLK,), jnp.int32),
            pltpu.VMEM((_BLK,), jnp.int32),
            pltpu.SemaphoreType.DMA,
        ],
    )
    return f(pdest)


def _gather_body(x_hbm, src_hbm, nsp_hbm, out_hbm,
                 src_v, nsp_v, tidx_v, ridx_v, data_v, zero_v, sem):
    cid = lax.axis_index("c")
    sid = lax.axis_index("s")
    wid = sid * 2 + cid
    a = wid * _BLK
    lanes = lax.iota(jnp.int32, 16)
    zf = jnp.zeros((16,), jnp.float32)

    pltpu.sync_copy(nsp_hbm.at[pl.ds(0, 16)], nsp_v)
    pltpu.sync_copy(src_hbm.at[pl.ds(a, _BLK)], src_v)
    nsp = nsp_v[pl.ds(0, 16)][0]
    count = jnp.clip(nsp - a, 0, _BLK)

    def _zb(t, c):
        zero_v[t // 32, pl.ds((t % 32) * 16, 16)] = zf
        return c

    lax.fori_loop(0, 512, _zb, 0)

    def _chunk(c, carry):
        v = count - c * 16

        @pl.when(v > 0)
        def _():
            raw = src_v[pl.ds(c * 16, 16)]
            safe = jnp.where(lanes < v, jnp.clip(raw, 0, _N_FRAMES - 1), 0)
            pltpu.async_copy(x_hbm.at[safe], data_v, sem).wait()

            @pl.when(v < 16)
            def _():
                def _zt(t, cc):
                    data_v[t // 32, pl.ds((t % 32) * 16, 16)] = zf
                    return cc
                lax.fori_loop(v * 32, 512, _zt, 0)

            pltpu.sync_copy(data_v, out_hbm.at[pl.ds(a + c * 16, 16)])

        @pl.when(v <= 0)
        def _():
            pltpu.sync_copy(zero_v, out_hbm.at[pl.ds(a + c * 16, 16)])

        return carry

    n_full = jnp.where(wid == _N_TILES - 1, _LAST_ROWS // 16, _GPT)
    lax.fori_loop(0, n_full, _chunk, 0)

    @pl.when(wid == _N_TILES - 1)
    def _():
        pltpu.sync_copy(src_hbm.at[pl.ds(_TAILDUP, 16)], tidx_v)
        ridx_v[...] = _TAIL0 + lanes
        v2 = count - _LTAIL0

        @pl.when(v2 > 0)
        def _():
            raw = tidx_v[pl.ds(0, 16)]
            safe = jnp.where(lanes < v2, jnp.clip(raw, 0, _N_FRAMES - 1), 0)
            pltpu.async_copy(x_hbm.at[safe], data_v, sem).wait()

        def _zt(t, cc):
            data_v[t // 32, pl.ds((t % 32) * 16, 16)] = zf
            return cc

        lax.fori_loop(jnp.clip(v2, 0, 16) * 32, 512, _zt, 0)
        pltpu.async_copy(data_v, out_hbm.at[ridx_v], sem).wait()


def _gather(x2d, src, nsp):
    mesh = plsc.VectorSubcoreMesh(core_axis_name="c", subcore_axis_name="s",
                                  num_cores=2, num_subcores=16)
    f = pl.kernel(
        _gather_body,
        out_type=jax.ShapeDtypeStruct((_N_FRAMES, _WINDOW), jnp.float32),
        mesh=mesh,
        scratch_types=[
            pltpu.VMEM((_BLK,), jnp.int32),
            pltpu.VMEM((16,), jnp.int32),
            pltpu.VMEM((16,), jnp.int32),
            pltpu.VMEM((16,), jnp.int32),
            pltpu.VMEM((16, _WINDOW), jnp.float32),
            pltpu.VMEM((16, _WINDOW), jnp.float32),
            pltpu.SemaphoreType.DMA,
        ],
    )
    return f(x2d, src, nsp)


def kernel(x, W, b):
    x2d = x[: _N_FRAMES * _WINDOW].reshape(_N_FRAMES, _WINDOW)
    pdest, nsp = _score(x2d, W, b)
    src = _scatter(pdest)
    out = _gather(x2d, src, nsp)
    return out.reshape(-1)


# speedup vs baseline: 19.0424x; 1.3475x over previous
import jax
import jax.numpy as jnp
from jax import lax
from jax.experimental import pallas as pl
from jax.experimental.pallas import tpu as pltpu
from jax.experimental.pallas import tpu_sc as plsc

_WINDOW = 512
_N_FRAMES = 18750
_BLK = 592
_N_TILES = 32
_N_PAD = _BLK * _N_TILES
_GPT = _BLK // 16
_SPT = 2 * _BLK + 16
_LAST_ROWS = _N_FRAMES - (_N_TILES - 1) * _BLK
_TAIL0 = _N_FRAMES - 16
_LTAIL0 = _TAIL0 - (_N_TILES - 1) * _BLK


def _score_body(x_ref, w_ref, b_ref, pdest_ref, bsum_ref, carry_ref,
                bvec_ref):
    k = pl.program_id(0)
    frames = x_ref[...]
    w = w_ref[...]
    logits = jnp.dot(frames, w, preferred_element_type=jnp.float32)
    logits = logits + b_ref[0]
    row = lax.broadcasted_iota(jnp.int32, (_BLK, 1), 0) + k * _BLK
    flag = jnp.where((logits > 0.0) & (row < _N_FRAMES), 1.0, 0.0)
    i = lax.broadcasted_iota(jnp.int32, (_BLK, _BLK), 0)
    j = lax.broadcasted_iota(jnp.int32, (_BLK, _BLK), 1)
    tri = jnp.where(i >= j, 1.0, 0.0)
    csum = jnp.dot(tri, flag, preferred_element_type=jnp.float32)

    @pl.when(k == 0)
    def _():
        carry_ref[0, 0] = 0.0
        bvec_ref[...] = jnp.zeros((1, 128), jnp.float32)

    carry = carry_ref[0, 0]
    dest = jnp.where(flag > 0.0, carry + csum, 0.0)
    pdest_ref[...] = dest.astype(jnp.int32).reshape(1, 1, _BLK)
    total = carry + jnp.max(csum)
    lane = lax.broadcasted_iota(jnp.int32, (1, 128), 1)
    bvec_ref[...] = jnp.where(lane == k, total, bvec_ref[...])
    bsum_ref[...] = bvec_ref[...].astype(jnp.int32).reshape(1, 1, 128)
    carry_ref[0, 0] = total


def _score(x2d, w, b):
    pdest, bsum = pl.pallas_call(
        _score_body,
        grid=(_N_TILES,),
        in_specs=[
            pl.BlockSpec((_BLK, _WINDOW), lambda k: (k, 0)),
            pl.BlockSpec((_WINDOW, 1), lambda k: (0, 0)),
            pl.BlockSpec(memory_space=pltpu.SMEM),
        ],
        out_specs=[
            pl.BlockSpec((1, 1, _BLK), lambda k: (k, 0, 0)),
            pl.BlockSpec((1, 1, 128), lambda k: (0, 0, 0)),
        ],
        out_shape=[
            jax.ShapeDtypeStruct((_N_TILES, 1, _BLK), jnp.int32),
            jax.ShapeDtypeStruct((1, 1, 128), jnp.int32),
        ],
        scratch_shapes=[pltpu.SMEM((1, 1), jnp.float32),
                        pltpu.VMEM((1, 128), jnp.float32)],
    )(x2d, w.reshape(_WINDOW, 1), b.reshape(1))
    return pdest.reshape(_N_PAD), bsum.reshape(128)


def _pack_body(x_hbm, pdest_hbm, bsum_hbm, out_hbm,
               pd_v, bs_v, pos_v, pos2_v, val_v, idx_v, tidx_v, ridx_v,
               data_v, zero_v, shared, sem):
    cid = lax.axis_index("c")
    sid = lax.axis_index("s")
    wid = sid * 2 + cid
    a = wid * _BLK
    bnd = a + _BLK
    base_sp = sid * _SPT
    lanes = lax.iota(jnp.int32, 16)
    zf = jnp.zeros((16,), jnp.float32)

    pltpu.sync_copy(bsum_hbm.at[pl.ds(0, 32)], bs_v)
    b0 = bs_v[pl.ds(0, 16)]
    b1 = bs_v[pl.ds(16, 16)]
    nsp = b1[15]
    count = jnp.clip(nsp - a, 0, _BLK)
    lo = jnp.int32(0)
    hi = jnp.int32(_N_TILES)

    def _zb(t, c):
        zero_v[t // 32, pl.ds((t % 32) * 16, 16)] = zf
        return c

    lax.fori_loop(0, 512, _zb, 0)

    def _blk(j, c):
        pltpu.sync_copy(pdest_hbm.at[pl.ds(j * _BLK, _BLK)], pd_v)

        def _g(g, cc):
            p = pd_v[pl.ds(g * 16, 16)]
            m = jnp.logical_and(p > a, p <= bnd)
            d = p - 1 - a
            k = g * 16 + lanes
            pos_v[pl.ds(g * 16, 16)] = base_sp + jnp.where(m, d, _BLK + k)
            m2 = jnp.logical_and(m, d >= _LTAIL0)
            pos2_v[pl.ds(g * 16, 16)] = base_sp + jnp.where(
                m2, 2 * _BLK + (d - _LTAIL0), _BLK + k)
            val_v[pl.ds(g * 16, 16)] = j * _BLK + k
            return cc

        lax.fori_loop(0, _GPT, _g, 0)
        pltpu.async_copy(val_v, shared.at[pos_v], sem).wait()

        @pl.when(wid == _N_TILES - 1)
        def _():
            pltpu.async_copy(val_v, shared.at[pos2_v], sem).wait()

        return c

    lax.fori_loop(lo, hi, _blk, 0)
    pltpu.sync_copy(shared.at[pl.ds(base_sp, _BLK)], idx_v)

    def _chunk(c, carry):
        v = count - c * 16

        @pl.when(v > 0)
        def _():
            raw = idx_v[pl.ds(c * 16, 16)]
            safe = jnp.where(lanes < v, jnp.clip(raw, 0, _N_FRAMES - 1), 0)
            pltpu.async_copy(x_hbm.at[safe], data_v, sem).wait()

            @pl.when(v < 16)
            def _():
                def _zt(t, cc):
                    data_v[t // 32, pl.ds((t % 32) * 16, 16)] = zf
                    return cc
                lax.fori_loop(v * 32, 512, _zt, 0)

            pltpu.sync_copy(data_v, out_hbm.at[pl.ds(a + c * 16, 16)])

        @pl.when(v <= 0)
        def _():
            pltpu.sync_copy(zero_v, out_hbm.at[pl.ds(a + c * 16, 16)])

        return carry

    n_full = jnp.where(wid == _N_TILES - 1, _LAST_ROWS // 16, _GPT)
    lax.fori_loop(0, n_full, _chunk, 0)

    @pl.when(wid == _N_TILES - 1)
    def _():
        pltpu.sync_copy(shared.at[pl.ds(base_sp + 2 * _BLK, 16)], tidx_v)
        ridx_v[...] = _TAIL0 + lanes
        v2 = count - _LTAIL0

        @pl.when(v2 > 0)
        def _():
            raw = tidx_v[pl.ds(0, 16)]
            safe = jnp.where(lanes < v2, jnp.clip(raw, 0, _N_FRAMES - 1), 0)
            pltpu.async_copy(x_hbm.at[safe], data_v, sem).wait()

        def _zt(t, cc):
            data_v[t // 32, pl.ds((t % 32) * 16, 16)] = zf
            return cc

        lax.fori_loop(jnp.clip(v2, 0, 16) * 32, 512, _zt, 0)
        pltpu.async_copy(data_v, out_hbm.at[ridx_v], sem).wait()


def _pack(x2d, pdest, bsum):
    mesh = plsc.VectorSubcoreMesh(core_axis_name="c", subcore_axis_name="s",
                                  num_cores=2, num_subcores=16)
    f = pl.kernel(
        _pack_body,
        out_type=jax.ShapeDtypeStruct((_N_FRAMES, _WINDOW), jnp.float32),
        mesh=mesh,
        scratch_types=[
            pltpu.VMEM((_BLK,), jnp.int32),
            pltpu.VMEM((32,), jnp.int32),
            pltpu.VMEM((_BLK,), jnp.int32),
            pltpu.VMEM((_BLK,), jnp.int32),
            pltpu.VMEM((_BLK,), jnp.int32),
            pltpu.VMEM((_BLK,), jnp.int32),
            pltpu.VMEM((16,), jnp.int32),
            pltpu.VMEM((16,), jnp.int32),
            pltpu.VMEM((16, _WINDOW), jnp.float32),
            pltpu.VMEM((16, _WINDOW), jnp.float32),
            pltpu.VMEM_SHARED((16 * _SPT,), jnp.int32),
            pltpu.SemaphoreType.DMA,
        ],
    )
    return f(x2d, pdest, bsum)


def kernel(x, W, b):
    x2d = x[: _N_FRAMES * _WINDOW].reshape(_N_FRAMES, _WINDOW)
    pdest, bsum = _score(x2d, W, b)
    out = _pack(x2d, pdest, bsum)
    return out.reshape(-1)


# speedup vs baseline: 22.0980x; 1.1605x over previous
import jax
import jax.numpy as jnp
from jax import lax
from jax.experimental import pallas as pl
from jax.experimental.pallas import tpu as pltpu
from jax.experimental.pallas import tpu_sc as plsc

_WINDOW = 512
_N_FRAMES = 18750
_BLK = 592
_N_TILES = 32
_N_PAD = _BLK * _N_TILES
_GPT = _BLK // 16
_SPT = 2 * _BLK + 16
_LAST_ROWS = _N_FRAMES - (_N_TILES - 1) * _BLK
_TAIL0 = _N_FRAMES - 16
_LTAIL0 = _TAIL0 - (_N_TILES - 1) * _BLK


def _score_body(x_ref, w_ref, b_ref, pdest_ref, bsum_ref, carry_ref,
                bvec_ref):
    k = pl.program_id(0)
    frames = x_ref[...]
    w = w_ref[...]
    logits = jnp.dot(frames, w, preferred_element_type=jnp.float32)
    logits = logits + b_ref[0]
    row = lax.broadcasted_iota(jnp.int32, (_BLK, 1), 0) + k * _BLK
    flag = jnp.where((logits > 0.0) & (row < _N_FRAMES), 1.0, 0.0)
    i = lax.broadcasted_iota(jnp.int32, (_BLK, _BLK), 0)
    j = lax.broadcasted_iota(jnp.int32, (_BLK, _BLK), 1)
    tri = jnp.where(i >= j, 1.0, 0.0)
    csum = jnp.dot(tri, flag, preferred_element_type=jnp.float32)

    @pl.when(k == 0)
    def _():
        carry_ref[0, 0] = 0.0
        bvec_ref[...] = jnp.zeros((1, 128), jnp.float32)

    carry = carry_ref[0, 0]
    dest = jnp.where(flag > 0.0, carry + csum, 0.0)
    pdest_ref[...] = dest.astype(jnp.int32).reshape(1, 1, _BLK)
    total = carry + jnp.max(csum)
    lane = lax.broadcasted_iota(jnp.int32, (1, 128), 1)
    bvec_ref[...] = jnp.where(lane == k, total, bvec_ref[...])
    bsum_ref[...] = bvec_ref[...].astype(jnp.int32).reshape(1, 1, 128)
    carry_ref[0, 0] = total


def _score(x2d, w, b):
    pdest, bsum = pl.pallas_call(
        _score_body,
        grid=(_N_TILES,),
        in_specs=[
            pl.BlockSpec((_BLK, _WINDOW), lambda k: (k, 0)),
            pl.BlockSpec((_WINDOW, 1), lambda k: (0, 0)),
            pl.BlockSpec(memory_space=pltpu.SMEM),
        ],
        out_specs=[
            pl.BlockSpec((1, 1, _BLK), lambda k: (k, 0, 0)),
            pl.BlockSpec((1, 1, 128), lambda k: (0, 0, 0)),
        ],
        out_shape=[
            jax.ShapeDtypeStruct((_N_TILES, 1, _BLK), jnp.int32),
            jax.ShapeDtypeStruct((1, 1, 128), jnp.int32),
        ],
        scratch_shapes=[pltpu.SMEM((1, 1), jnp.float32),
                        pltpu.VMEM((1, 128), jnp.float32)],
    )(x2d, w.reshape(_WINDOW, 1), b.reshape(1))
    return pdest.reshape(_N_PAD), bsum.reshape(128)


def _pack_body(x_hbm, pdest_hbm, bsum_hbm, out_hbm,
               pd_v, bs_v, pos_v, pos2_v, val_v, idx_v, tidx_v, ridx_v,
               data_v, zero_v, shared, sem):
    cid = lax.axis_index("c")
    sid = lax.axis_index("s")
    wid = sid * 2 + cid
    a = wid * _BLK
    bnd = a + _BLK
    base_sp = sid * _SPT
    lanes = lax.iota(jnp.int32, 16)
    zf = jnp.zeros((16,), jnp.float32)

    pltpu.sync_copy(bsum_hbm.at[pl.ds(0, 32)], bs_v)
    b0 = bs_v[pl.ds(0, 16)]
    b1 = bs_v[pl.ds(16, 16)]
    nsp = b1[15]
    count = jnp.clip(nsp - a, 0, _BLK)
    lo = jnp.int32(0)
    hi = jnp.int32(1)
    for jj in range(16):
        lo = lo + jnp.where(b0[jj] <= a, 1, 0)
        hi = hi + jnp.where(b0[jj] < bnd, 1, 0)
    for jj in range(16):
        lo = lo + jnp.where(b1[jj] <= a, 1, 0)
        if jj <= 14:
            hi = hi + jnp.where(b1[jj] < bnd, 1, 0)
    hi = jnp.minimum(hi, _N_TILES)
    lo = jnp.minimum(lo, hi)

    def _zb(t, c):
        zero_v[t // 32, pl.ds((t % 32) * 16, 16)] = zf
        return c

    lax.fori_loop(0, 512, _zb, 0)

    def _blk(j, c):
        pltpu.sync_copy(pdest_hbm.at[pl.ds(j * _BLK, _BLK)], pd_v)

        def _g(g, cc):
            p = pd_v[pl.ds(g * 16, 16)]
            m = jnp.logical_and(p > a, p <= bnd)
            d = p - 1 - a
            k = g * 16 + lanes
            pos_v[pl.ds(g * 16, 16)] = base_sp + jnp.where(m, d, _BLK + k)
            m2 = jnp.logical_and(m, d >= _LTAIL0)
            pos2_v[pl.ds(g * 16, 16)] = base_sp + jnp.where(
                m2, 2 * _BLK + (d - _LTAIL0), _BLK + k)
            val_v[pl.ds(g * 16, 16)] = j * _BLK + k
            return cc

        lax.fori_loop(0, _GPT, _g, 0)
        pltpu.async_copy(val_v, shared.at[pos_v], sem).wait()

        @pl.when(wid == _N_TILES - 1)
        def _():
            pltpu.async_copy(val_v, shared.at[pos2_v], sem).wait()

        return c

    lax.fori_loop(lo, hi, _blk, 0)
    pltpu.sync_copy(shared.at[pl.ds(base_sp, _BLK)], idx_v)

    def _chunk(c, carry):
        v = count - c * 16

        @pl.when(v > 0)
        def _():
            raw = idx_v[pl.ds(c * 16, 16)]
            safe = jnp.where(lanes < v, jnp.clip(raw, 0, _N_FRAMES - 1), 0)
            pltpu.async_copy(x_hbm.at[safe], data_v, sem).wait()

            @pl.when(v < 16)
            def _():
                def _zt(t, cc):
                    data_v[t // 32, pl.ds((t % 32) * 16, 16)] = zf
                    return cc
                lax.fori_loop(v * 32, 512, _zt, 0)

            pltpu.sync_copy(data_v, out_hbm.at[pl.ds(a + c * 16, 16)])

        @pl.when(v <= 0)
        def _():
            pltpu.sync_copy(zero_v, out_hbm.at[pl.ds(a + c * 16, 16)])

        return carry

    n_full = jnp.where(wid == _N_TILES - 1, _LAST_ROWS // 16, _GPT)
    lax.fori_loop(0, n_full, _chunk, 0)

    @pl.when(wid == _N_TILES - 1)
    def _():
        pltpu.sync_copy(shared.at[pl.ds(base_sp + 2 * _BLK, 16)], tidx_v)
        ridx_v[...] = _TAIL0 + lanes
        v2 = count - _LTAIL0

        @pl.when(v2 > 0)
        def _():
            raw = tidx_v[pl.ds(0, 16)]
            safe = jnp.where(lanes < v2, jnp.clip(raw, 0, _N_FRAMES - 1), 0)
            pltpu.async_copy(x_hbm.at[safe], data_v, sem).wait()

        def _zt(t, cc):
            data_v[t // 32, pl.ds((t % 32) * 16, 16)] = zf
            return cc

        lax.fori_loop(jnp.clip(v2, 0, 16) * 32, 512, _zt, 0)
        pltpu.async_copy(data_v, out_hbm.at[ridx_v], sem).wait()


def _pack(x2d, pdest, bsum):
    mesh = plsc.VectorSubcoreMesh(core_axis_name="c", subcore_axis_name="s",
                                  num_cores=2, num_subcores=16)
    f = pl.kernel(
        _pack_body,
        out_type=jax.ShapeDtypeStruct((_N_FRAMES, _WINDOW), jnp.float32),
        mesh=mesh,
        scratch_types=[
            pltpu.VMEM((_BLK,), jnp.int32),
            pltpu.VMEM((32,), jnp.int32),
            pltpu.VMEM((_BLK,), jnp.int32),
            pltpu.VMEM((_BLK,), jnp.int32),
            pltpu.VMEM((_BLK,), jnp.int32),
            pltpu.VMEM((_BLK,), jnp.int32),
            pltpu.VMEM((16,), jnp.int32),
            pltpu.VMEM((16,), jnp.int32),
            pltpu.VMEM((16, _WINDOW), jnp.float32),
            pltpu.VMEM((16, _WINDOW), jnp.float32),
            pltpu.VMEM_SHARED((16 * _SPT,), jnp.int32),
            pltpu.SemaphoreType.DMA,
        ],
    )
    return f(x2d, pdest, bsum)


def kernel(x, W, b):
    x2d = x[: _N_FRAMES * _WINDOW].reshape(_N_FRAMES, _WINDOW)
    pdest, bsum = _score(x2d, W, b)
    out = _pack(x2d, pdest, bsum)
    return out.reshape(-1)


# speedup vs baseline: 25.1624x; 1.1387x over previous
import jax
import jax.numpy as jnp
from jax import lax
from jax.experimental import pallas as pl
from jax.experimental.pallas import tpu as pltpu
from jax.experimental.pallas import tpu_sc as plsc

_WINDOW = 512
_N_FRAMES = 18750
_BLK = 592
_N_TILES = 32
_N_PAD = _BLK * _N_TILES
_GPT = _BLK // 16
_SPT = 2 * _BLK + 16
_LAST_ROWS = _N_FRAMES - (_N_TILES - 1) * _BLK
_TAIL0 = _N_FRAMES - 16
_LTAIL0 = _TAIL0 - (_N_TILES - 1) * _BLK


def _score_body(x_ref, w_ref, b_ref, pdest_ref, bsum_ref, carry_ref,
                bvec_ref):
    k = pl.program_id(0)
    frames = x_ref[...]
    w = w_ref[...]
    logits = jnp.dot(frames, w, preferred_element_type=jnp.float32)
    logits = logits + b_ref[0]
    row = lax.broadcasted_iota(jnp.int32, (_BLK, 1), 0) + k * _BLK
    flag = jnp.where((logits > 0.0) & (row < _N_FRAMES), 1.0, 0.0)
    i = lax.broadcasted_iota(jnp.int32, (_BLK, _BLK), 0)
    j = lax.broadcasted_iota(jnp.int32, (_BLK, _BLK), 1)
    tri = jnp.where(i >= j, 1.0, 0.0)
    csum = jnp.dot(tri, flag, preferred_element_type=jnp.float32)

    @pl.when(k == 0)
    def _():
        carry_ref[0, 0] = 0.0
        bvec_ref[...] = jnp.zeros((1, 128), jnp.float32)

    carry = carry_ref[0, 0]
    dest = jnp.where(flag > 0.0, carry + csum, 0.0)
    pdest_ref[...] = dest.astype(jnp.int32).reshape(1, 1, _BLK)
    total = carry + jnp.max(csum)
    lane = lax.broadcasted_iota(jnp.int32, (1, 128), 1)
    bvec_ref[...] = jnp.where(lane == k, total, bvec_ref[...])
    bsum_ref[...] = bvec_ref[...].astype(jnp.int32).reshape(1, 1, 128)
    carry_ref[0, 0] = total


def _score(x2d, w, b):
    pdest, bsum = pl.pallas_call(
        _score_body,
        grid=(_N_TILES,),
        in_specs=[
            pl.BlockSpec((_BLK, _WINDOW), lambda k: (k, 0)),
            pl.BlockSpec((_WINDOW, 1), lambda k: (0, 0)),
            pl.BlockSpec(memory_space=pltpu.SMEM),
        ],
        out_specs=[
            pl.BlockSpec((1, 1, _BLK), lambda k: (k, 0, 0)),
            pl.BlockSpec((1, 1, 128), lambda k: (0, 0, 0)),
        ],
        out_shape=[
            jax.ShapeDtypeStruct((_N_TILES, 1, _BLK), jnp.int32),
            jax.ShapeDtypeStruct((1, 1, 128), jnp.int32),
        ],
        scratch_shapes=[pltpu.SMEM((1, 1), jnp.float32),
                        pltpu.VMEM((1, 128), jnp.float32)],
    )(x2d, w.reshape(_WINDOW, 1), b.reshape(1))
    return pdest.reshape(_N_PAD), bsum.reshape(128)


def _pack_body(x_hbm, pdest_hbm, bsum_hbm, out_hbm,
               pd_v, bs_v, pos_v, pos2_v, val_v, idx_v, tidx_v, ridx_v,
               data_v, data2_v, zero_v, shared, sem):
    cid = lax.axis_index("c")
    sid = lax.axis_index("s")
    wid = sid * 2 + cid
    a = wid * _BLK
    bnd = a + _BLK
    base_sp = sid * _SPT
    lanes = lax.iota(jnp.int32, 16)
    zf = jnp.zeros((16,), jnp.float32)

    pltpu.sync_copy(bsum_hbm.at[pl.ds(0, 32)], bs_v)
    b0 = bs_v[pl.ds(0, 16)]
    b1 = bs_v[pl.ds(16, 16)]
    nsp = b1[15]
    count = jnp.clip(nsp - a, 0, _BLK)
    lo = jnp.int32(0)
    hi = jnp.int32(1)
    for jj in range(16):
        lo = lo + jnp.where(b0[jj] <= a, 1, 0)
        hi = hi + jnp.where(b0[jj] < bnd, 1, 0)
    for jj in range(16):
        lo = lo + jnp.where(b1[jj] <= a, 1, 0)
        if jj <= 14:
            hi = hi + jnp.where(b1[jj] < bnd, 1, 0)
    hi = jnp.minimum(hi, _N_TILES)
    lo = jnp.minimum(lo, hi)

    def _blk(j, c):
        pltpu.sync_copy(pdest_hbm.at[pl.ds(j * _BLK, _BLK)], pd_v)

        def _g(g, cc):
            p = pd_v[pl.ds(g * 16, 16)]
            m = jnp.logical_and(p > a, p <= bnd)
            d = p - 1 - a
            k = g * 16 + lanes
            pos_v[pl.ds(g * 16, 16)] = base_sp + jnp.where(m, d, _BLK + k)
            m2 = jnp.logical_and(m, d >= _LTAIL0)
            pos2_v[pl.ds(g * 16, 16)] = base_sp + jnp.where(
                m2, 2 * _BLK + (d - _LTAIL0), _BLK + k)
            val_v[pl.ds(g * 16, 16)] = j * _BLK + k
            return cc

        lax.fori_loop(0, _GPT, _g, 0)
        pltpu.async_copy(val_v, shared.at[pos_v], sem).wait()

        @pl.when(wid == _N_TILES - 1)
        def _():
            pltpu.async_copy(val_v, shared.at[pos2_v], sem).wait()

        return c

    lax.fori_loop(lo, hi, _blk, 0)
    pltpu.sync_copy(shared.at[pl.ds(base_sp, _BLK)], idx_v)

    n_full = jnp.where(wid == _N_TILES - 1, _LAST_ROWS // 16, _GPT)

    def _start(c, buf):
        @pl.when(jnp.logical_and(c < n_full, count > c * 16))
        def _():
            v = count - c * 16
            raw = idx_v[pl.ds(c * 16, 16)]
            safe = jnp.where(lanes < v, jnp.clip(raw, 0, _N_FRAMES - 1), 0)
            pltpu.async_copy(x_hbm.at[safe], buf, sem)

    def _finish(c, buf):
        v = count - c * 16

        @pl.when(jnp.logical_and(c < n_full, v > 0))
        def _():
            pltpu.make_async_copy(x_hbm.at[pl.ds(0, 16)], buf, sem).wait()

            @pl.when(v < 16)
            def _():
                def _zt(t, cc):
                    buf[t // 32, pl.ds((t % 32) * 16, 16)] = zf
                    return cc
                lax.fori_loop(v * 32, 512, _zt, 0)

            pltpu.sync_copy(buf, out_hbm.at[pl.ds(a + c * 16, 16)])

        @pl.when(jnp.logical_and(c < n_full, v <= 0))
        def _():
            pltpu.sync_copy(zero_v, out_hbm.at[pl.ds(a + c * 16, 16)])

    _start(jnp.int32(0), data_v)

    def _zb(t, c):
        zero_v[t // 32, pl.ds((t % 32) * 16, 16)] = zf
        return c

    lax.fori_loop(0, 512, _zb, 0)

    def _pair(c2, carry):
        c = c2 * 2
        _start(c + 1, data2_v)
        _finish(c, data_v)
        _start(c + 2, data_v)
        _finish(c + 1, data2_v)
        return carry

    lax.fori_loop(0, (_GPT + 1) // 2, _pair, 0)

    @pl.when(wid == _N_TILES - 1)
    def _():
        pltpu.sync_copy(shared.at[pl.ds(base_sp + 2 * _BLK, 16)], tidx_v)
        ridx_v[...] = _TAIL0 + lanes
        v2 = count - _LTAIL0

        @pl.when(v2 > 0)
        def _():
            raw = tidx_v[pl.ds(0, 16)]
            safe = jnp.where(lanes < v2, jnp.clip(raw, 0, _N_FRAMES - 1), 0)
            pltpu.async_copy(x_hbm.at[safe], data_v, sem).wait()

        def _zt(t, cc):
            data_v[t // 32, pl.ds((t % 32) * 16, 16)] = zf
            return cc

        lax.fori_loop(jnp.clip(v2, 0, 16) * 32, 512, _zt, 0)
        pltpu.async_copy(data_v, out_hbm.at[ridx_v], sem).wait()


def _pack(x2d, pdest, bsum):
    mesh = plsc.VectorSubcoreMesh(core_axis_name="c", subcore_axis_name="s",
                                  num_cores=2, num_subcores=16)
    f = pl.kernel(
        _pack_body,
        out_type=jax.ShapeDtypeStruct((_N_FRAMES, _WINDOW), jnp.float32),
        mesh=mesh,
        scratch_types=[
            pltpu.VMEM((_BLK,), jnp.int32),
            pltpu.VMEM((32,), jnp.int32),
            pltpu.VMEM((_BLK,), jnp.int32),
            pltpu.VMEM((_BLK,), jnp.int32),
            pltpu.VMEM((_BLK,), jnp.int32),
            pltpu.VMEM((_BLK,), jnp.int32),
            pltpu.VMEM((16,), jnp.int32),
            pltpu.VMEM((16,), jnp.int32),
            pltpu.VMEM((16, _WINDOW), jnp.float32),
            pltpu.VMEM((16, _WINDOW), jnp.float32),
            pltpu.VMEM((16, _WINDOW), jnp.float32),
            pltpu.VMEM_SHARED((16 * _SPT,), jnp.int32),
            pltpu.SemaphoreType.DMA,
        ],
    )
    return f(x2d, pdest, bsum)


def kernel(x, W, b):
    x2d = x[: _N_FRAMES * _WINDOW].reshape(_N_FRAMES, _WINDOW)
    pdest, bsum = _score(x2d, W, b)
    out = _pack(x2d, pdest, bsum)
    return out.reshape(-1)


# speedup vs baseline: 31.6473x; 1.2577x over previous
import jax
import jax.numpy as jnp
from jax import lax
from jax.experimental import pallas as pl
from jax.experimental.pallas import tpu as pltpu
from jax.experimental.pallas import tpu_sc as plsc

_WINDOW = 512
_N_FRAMES = 18750
_N_ROWS = _N_FRAMES * 4
_BLK = 592
_N_TILES = 32
_N_PAD = _BLK * _N_TILES
_GPT = _BLK // 16
_SPT = 2 * _BLK + 4 * _BLK
_LAST_ROWS = _N_FRAMES - (_N_TILES - 1) * _BLK
_LAST_FULL = _LAST_ROWS // 16
_TAILF = _LAST_ROWS - _LAST_FULL * 16


def _score_body(x_ref, w_ref, b_ref, pdest_ref, bsum_ref, carry_ref,
                bvec_ref):
    k = pl.program_id(0)
    frames = x_ref[...]
    w = w_ref[...]
    logits = jnp.dot(frames, w, preferred_element_type=jnp.float32)
    logits = logits + b_ref[0]
    row = lax.broadcasted_iota(jnp.int32, (_BLK, 1), 0) + k * _BLK
    flag = jnp.where((logits > 0.0) & (row < _N_FRAMES), 1.0, 0.0)
    i = lax.broadcasted_iota(jnp.int32, (_BLK, _BLK), 0)
    j = lax.broadcasted_iota(jnp.int32, (_BLK, _BLK), 1)
    tri = jnp.where(i >= j, 1.0, 0.0)
    csum = jnp.dot(tri, flag, preferred_element_type=jnp.float32)

    @pl.when(k == 0)
    def _():
        carry_ref[0, 0] = 0.0
        bvec_ref[...] = jnp.zeros((1, 128), jnp.float32)

    carry = carry_ref[0, 0]
    dest = jnp.where(flag > 0.0, carry + csum, 0.0)
    pdest_ref[...] = dest.astype(jnp.int32).reshape(1, 1, _BLK)
    total = carry + jnp.max(csum)
    lane = lax.broadcasted_iota(jnp.int32, (1, 128), 1)
    bvec_ref[...] = jnp.where(lane == k, total, bvec_ref[...])
    bsum_ref[...] = bvec_ref[...].astype(jnp.int32).reshape(1, 1, 128)
    carry_ref[0, 0] = total


def _score(x2d, w, b):
    pdest, bsum = pl.pallas_call(
        _score_body,
        grid=(_N_TILES,),
        in_specs=[
            pl.BlockSpec((_BLK, _WINDOW), lambda k: (k, 0)),
            pl.BlockSpec((_WINDOW, 1), lambda k: (0, 0)),
            pl.BlockSpec(memory_space=pltpu.SMEM),
        ],
        out_specs=[
            pl.BlockSpec((1, 1, _BLK), lambda k: (k, 0, 0)),
            pl.BlockSpec((1, 1, 128), lambda k: (0, 0, 0)),
        ],
        out_shape=[
            jax.ShapeDtypeStruct((_N_TILES, 1, _BLK), jnp.int32),
            jax.ShapeDtypeStruct((1, 1, 128), jnp.int32),
        ],
        scratch_shapes=[pltpu.SMEM((1, 1), jnp.float32),
                        pltpu.VMEM((1, 128), jnp.float32)],
    )(x2d, w.reshape(_WINDOW, 1), b.reshape(1))
    return pdest.reshape(_N_PAD), bsum.reshape(128)


def _pack_body(x_hbm, pdest_hbm, bsum_hbm, out_hbm,
               pd_v, bs_v, pos_v, val_v, idx_v, idx4_v,
               data_v, data2_v, zero_v, shared, sem):
    cid = lax.axis_index("c")
    sid = lax.axis_index("s")
    wid = sid * 2 + cid
    a = wid * _BLK
    bnd = a + _BLK
    base_sp = sid * _SPT
    lanes = lax.iota(jnp.int32, 16)
    zf = jnp.zeros((16,), jnp.float32)

    pltpu.sync_copy(bsum_hbm.at[pl.ds(0, 32)], bs_v)
    b0 = bs_v[pl.ds(0, 16)]
    b1 = bs_v[pl.ds(16, 16)]
    nsp = b1[15]
    count = jnp.clip(nsp - a, 0, _BLK)
    lo = jnp.int32(0)
    hi = jnp.int32(1)
    for jj in range(16):
        lo = lo + jnp.where(b0[jj] <= a, 1, 0)
        hi = hi + jnp.where(b0[jj] < bnd, 1, 0)
    for jj in range(16):
        lo = lo + jnp.where(b1[jj] <= a, 1, 0)
        if jj <= 14:
            hi = hi + jnp.where(b1[jj] < bnd, 1, 0)
    hi = jnp.minimum(hi, _N_TILES)
    lo = jnp.minimum(lo, hi)

    def _blk(j, c):
        pltpu.sync_copy(pdest_hbm.at[pl.ds(j * _BLK, _BLK)], pd_v)

        def _g(g, cc):
            p = pd_v[pl.ds(g * 16, 16)]
            m = jnp.logical_and(p > a, p <= bnd)
            d = p - 1 - a
            k = g * 16 + lanes
            pos_v[pl.ds(g * 16, 16)] = base_sp + jnp.where(m, d, _BLK + k)
            val_v[pl.ds(g * 16, 16)] = j * _BLK + k
            return cc

        lax.fori_loop(0, _GPT, _g, 0)
        pltpu.async_copy(val_v, shared.at[pos_v], sem).wait()
        return c

    lax.fori_loop(lo, hi, _blk, 0)
    pltpu.sync_copy(shared.at[pl.ds(base_sp, _BLK)], idx_v)

    for q in range(4):
        def _e(g, cc):
            fr = idx_v[pl.ds(g * 16, 16)]
            k = g * 16 + lanes
            safe = jnp.where(k < count, jnp.clip(fr, 0, _N_FRAMES - 1), 0)
            pos_v[pl.ds(g * 16, 16)] = base_sp + 2 * _BLK + 4 * k + q
            val_v[pl.ds(g * 16, 16)] = 4 * safe + q
            return cc

        lax.fori_loop(0, _GPT, _e, 0)
        pltpu.async_copy(val_v, shared.at[pos_v], sem).wait()

    pltpu.sync_copy(shared.at[pl.ds(base_sp + 2 * _BLK, 4 * _BLK)], idx4_v)

    n_full = jnp.where(wid == _N_TILES - 1, _LAST_FULL, _GPT)

    def _start(c, buf):
        @pl.when(jnp.logical_and(c < n_full, count > c * 16))
        def _():
            pltpu.async_copy(x_hbm.at[idx4_v.at[pl.ds(c * 64, 64)]], buf,
                             sem)

    def _finish(c, buf):
        v = count - c * 16

        @pl.when(jnp.logical_and(c < n_full, v > 0))
        def _():
            pltpu.make_async_copy(x_hbm.at[pl.ds(0, 64)], buf, sem).wait()

            @pl.when(v < 16)
            def _():
                def _zt(t, cc):
                    buf[t // 8, pl.ds((t % 8) * 16, 16)] = zf
                    return cc
                lax.fori_loop(v * 32, 512, _zt, 0)

            pltpu.sync_copy(buf, out_hbm.at[pl.ds((a + c * 16) * 4, 64)])

        @pl.when(jnp.logical_and(c < n_full, v <= 0))
        def _():
            pltpu.sync_copy(zero_v, out_hbm.at[pl.ds((a + c * 16) * 4, 64)])

    _start(jnp.int32(0), data_v)

    def _zb(t, c):
        zero_v[t // 8, pl.ds((t % 8) * 16, 16)] = zf
        return c

    lax.fori_loop(0, 512, _zb, 0)

    def _pair(c2, carry):
        c = c2 * 2
        _start(c + 1, data2_v)
        _finish(c, data_v)
        _start(c + 2, data_v)
        _finish(c + 1, data2_v)
        return carry

    lax.fori_loop(0, (_GPT + 1) // 2, _pair, 0)

    @pl.when(wid == _N_TILES - 1)
    def _():
        v2 = count - _LAST_FULL * 16

        @pl.when(v2 > 0)
        def _():
            pltpu.async_copy(
                x_hbm.at[idx4_v.at[pl.ds(_LAST_FULL * 64, 4 * _TAILF)]],
                data_v.at[pl.ds(0, 4 * _TAILF)], sem).wait()

        def _zt(t, cc):
            data_v[t // 8, pl.ds((t % 8) * 16, 16)] = zf
            return cc

        lax.fori_loop(jnp.clip(v2, 0, _TAILF) * 32, 4 * _TAILF * 8, _zt, 0)
        pltpu.sync_copy(data_v.at[pl.ds(0, 4 * _TAILF)],
                        out_hbm.at[pl.ds(_N_ROWS - 4 * _TAILF, 4 * _TAILF)])


def _pack(x4, pdest, bsum):
    mesh = plsc.VectorSubcoreMesh(core_axis_name="c", subcore_axis_name="s",
                                  num_cores=2, num_subcores=16)
    f = pl.kernel(
        _pack_body,
        out_type=jax.ShapeDtypeStruct((_N_ROWS, 128), jnp.float32),
        mesh=mesh,
        scratch_types=[
            pltpu.VMEM((_BLK,), jnp.int32),
            pltpu.VMEM((32,), jnp.int32),
            pltpu.VMEM((_BLK,), jnp.int32),
            pltpu.VMEM((_BLK,), jnp.int32),
            pltpu.VMEM((_BLK,), jnp.int32),
            pltpu.VMEM((4 * _BLK,), jnp.int32),
            pltpu.VMEM((64, 128), jnp.float32),
            pltpu.VMEM((64, 128), jnp.float32),
            pltpu.VMEM((64, 128), jnp.float32),
            pltpu.VMEM_SHARED((16 * _SPT,), jnp.int32),
            pltpu.SemaphoreType.DMA,
        ],
    )
    return f(x4, pdest, bsum)


def kernel(x, W, b):
    x2d = x.reshape(_N_FRAMES, _WINDOW)
    x4 = x.reshape(_N_ROWS, 128)
    pdest, bsum = _score(x2d, W, b)
    out = _pack(x4, pdest, bsum)
    return out.reshape(-1)
